# baseline (speedup 1.0000x reference)
import functools
import math

import jax
import jax.numpy as jnp
from jax import lax
from jax.experimental import pallas as pl
from jax.experimental.pallas import tpu as pltpu
from jax.experimental.pallas import tpu_sc as plsc

N = 100000
DH = 128
NSEG = 1000

BN = 2048
GRID = 49
NPAD = BN * GRID

NT = 16
PER_TILE = NPAD // NT
CH = 128
NCH = PER_TILE // CH
NBIN = 1024
SLICE = NBIN // NT

_HALF_PI = 0.5 * math.pi

_COS_C = (0.9999999672669706, -0.49999926887023616, 0.04166409103912293,
          -0.0013857419130200166, 2.3237578000323555e-05)
_SIN_C = (0.9999999827780375, -0.1666665151722809, 0.008332963963586328,
          -0.00019804751716372138, 2.5981027733101847e-06)


def _poly4(u, k):
    acc = jnp.float32(k[4])
    for i in (3, 2, 1, 0):
        acc = acc * u + jnp.float32(k[i])
    return acc


def _trig_body(t2_ref, c_ref, s_ref):
    a = _HALF_PI * t2_ref[...]
    u = a * a
    c_ref[...] = _HALF_PI * _poly4(u, _COS_C)
    s_ref[...] = (_HALF_PI * a) * _poly4(u, _SIN_C)


def _trig(t2):
    shp = jax.ShapeDtypeStruct((NPAD // 128, 128), jnp.float32)
    return pl.pallas_call(_trig_body, out_shape=(shp, shp))(t2)


def _loss_body(c_ref, s_ref, dx_ref, dh_ref, zx_ref, zh_ref, x_ref, h_ref,
               out_ref):
    c = c_ref[...]
    s = s_ref[...]
    hp = dh_ref[...] - (c * h_ref[...] - s * zh_ref[...])
    xp = dx_ref[...] - (x_ref[...] - zx_ref[...])
    out_ref[...] = (
        jnp.dot(hp * hp, jnp.ones((DH, 1), jnp.float32),
                preferred_element_type=jnp.float32)
        + jnp.dot(xp * xp, jnp.ones((3, 1), jnp.float32),
                  preferred_element_type=jnp.float32))


def _node_loss(c, s, dx_t, dh_t, z_x, z_h, x, h):
    row = lambda i: (i, 0)
    return pl.pallas_call(
        _loss_body,
        grid=(GRID,),
        in_specs=[
            pl.BlockSpec((BN, 1), row),
            pl.BlockSpec((BN, 1), row),
            pl.BlockSpec((BN, 3), row),
            pl.BlockSpec((BN, DH), row),
            pl.BlockSpec((BN, 3), row),
            pl.BlockSpec((BN, DH), row),
            pl.BlockSpec((BN, 3), row),
            pl.BlockSpec((BN, DH), row),
        ],
        out_specs=pl.BlockSpec((BN, 1), row),
        out_shape=jax.ShapeDtypeStruct((NPAD, 1), jnp.float32),
    )(c, s, dx_t, dh_t, z_x, z_h, x, h)


def _seg_body(loss_hbm, idx_hbm, out_hbm,
              vals_v, idx_v, ones_v, zeros_v, s_v, c_v, r_v,
              sums_sh, cnts_sh, sem):
    tid = lax.axis_index("s")
    base = tid * PER_TILE

    pltpu.sync_copy(loss_hbm.at[pl.ds(base, PER_TILE)], vals_v)
    pltpu.sync_copy(idx_hbm.at[tid], idx_v)

    for i in range(CH // 16):
        ones_v[pl.ds(16 * i, 16)] = jnp.full((16,), 1.0, jnp.float32)
    for i in range(SLICE // 16):
        zeros_v[pl.ds(16 * i, 16)] = jnp.zeros((16,), jnp.float32)

    pltpu.sync_copy(zeros_v, sums_sh.at[pl.ds(tid * SLICE, SLICE)])
    pltpu.sync_copy(zeros_v, cnts_sh.at[pl.ds(tid * SLICE, SLICE)])
    plsc.subcore_barrier()

    G = 7

    def group(g, carry):
        def fire(j, c2):
            jj = g * G + j
            pltpu.async_copy(vals_v.at[pl.ds(jj * CH, CH)],
                             sums_sh.at[idx_v.at[jj]], sem, add=True)
            pltpu.async_copy(ones_v, cnts_sh.at[idx_v.at[jj]], sem, add=True)
            return c2

        def drain(j, c2):
            pltpu.make_async_copy(ones_v, sums_sh.at[idx_v.at[0]], sem).wait()
            pltpu.make_async_copy(ones_v, cnts_sh.at[idx_v.at[0]], sem).wait()
            return c2

        lax.fori_loop(0, G, fire, 0)
        lax.fori_loop(0, G, drain, 0)
        return carry

    lax.fori_loop(0, NCH // G, group, 0)
    plsc.subcore_barrier()

    pltpu.sync_copy(sums_sh.at[pl.ds(tid * SLICE, SLICE)], s_v)
    pltpu.sync_copy(cnts_sh.at[pl.ds(tid * SLICE, SLICE)], c_v)
    for i in range(SLICE // 16):
        sl = pl.ds(16 * i, 16)
        r_v[sl] = s_v[sl] / jnp.maximum(c_v[sl], 1.0)

    @pl.when(tid < NT - 1)
    def _():
        pltpu.sync_copy(r_v, out_hbm.at[pl.ds(tid * SLICE, SLICE)])

    @pl.when(tid == NT - 1)
    def _():
        rem = NSEG - (NT - 1) * SLICE
        pltpu.sync_copy(r_v.at[pl.ds(0, rem)],
                        out_hbm.at[pl.ds((NT - 1) * SLICE, rem)])


def _seg_call(loss, idx):
    return pl.kernel(
        _seg_body,
        out_type=jax.ShapeDtypeStruct((NSEG,), jnp.float32),
        mesh=plsc.VectorSubcoreMesh(core_axis_name="c", subcore_axis_name="s",
                                    num_cores=1, num_subcores=NT),
        scratch_types=[
        pltpu.VMEM((PER_TILE,), jnp.float32),
        pltpu.VMEM((NCH, CH), jnp.int32),
        pltpu.VMEM((CH,), jnp.float32),
        pltpu.VMEM((SLICE,), jnp.float32),
        pltpu.VMEM((SLICE,), jnp.float32),
        pltpu.VMEM((SLICE,), jnp.float32),
        pltpu.VMEM((SLICE,), jnp.float32),
        pltpu.VMEM_SHARED((NBIN,), jnp.float32),
        pltpu.VMEM_SHARED((NBIN,), jnp.float32),
        pltpu.SemaphoreType.DMA,
        ],
    )(loss, idx)


def kernel(t, dx_t, dh_t, z_x, z_h, x, h, segment_ids):
    t2 = jnp.pad(t.reshape(N), (0, NPAD - N)).reshape(NPAD // 128, 128)
    c, s = _trig(t2)
    c = c.reshape(NPAD, 1)
    s = s.reshape(NPAD, 1)
    loss = _node_loss(c, s, dx_t, dh_t, z_x, z_h, x, h).reshape(NPAD)
    seg = segment_ids.astype(jnp.int32)
    pad = jnp.full((NPAD - N,), NSEG, jnp.int32)
    idx = jnp.concatenate([seg, pad]).reshape(NT, NCH, CH)
    return _seg_call(loss, idx)


# speedup vs baseline: 1.4929x; 1.4929x over previous
import functools
import math

import jax
import jax.numpy as jnp
from jax import lax
from jax.experimental import pallas as pl
from jax.experimental.pallas import tpu as pltpu
from jax.experimental.pallas import tpu_sc as plsc

N = 100000
DH = 128
NSEG = 1000

BN = 2048
GRID = 49
NPAD = BN * GRID

NT = 16
PER_TILE = NPAD // NT
CH = 128
NCH = PER_TILE // CH
NBIN = 1024
SLICE = NBIN // NT

_HALF_PI = 0.5 * math.pi

_COS_C = (0.9999999672669706, -0.49999926887023616, 0.04166409103912293,
          -0.0013857419130200166, 2.3237578000323555e-05)
_SIN_C = (0.9999999827780375, -0.1666665151722809, 0.008332963963586328,
          -0.00019804751716372138, 2.5981027733101847e-06)


def _poly4(u, k):
    acc = jnp.float32(k[4])
    for i in (3, 2, 1, 0):
        acc = acc * u + jnp.float32(k[i])
    return acc


def _rowsum_t(prod, k):
    ones = jnp.ones((k, 1), jnp.float32)
    return jax.lax.dot_general(
        ones, prod, (((0,), (1,)), ((), ())),
        preferred_element_type=jnp.float32)


def _loss_body(t_ref, dx_ref, dh_ref, zx_ref, zh_ref, x_ref, h_ref, out_ref):
    a = _HALF_PI * t_ref[0]
    u = a * a
    c = _HALF_PI * _poly4(u, _COS_C)
    s = (_HALF_PI * a) * _poly4(u, _SIN_C)

    dh = dh_ref[...]
    hh = h_ref[...]
    zh = zh_ref[...]
    A = _rowsum_t(dh * dh, DH)
    B = _rowsum_t(dh * hh, DH)
    C = _rowsum_t(dh * zh, DH)
    D = _rowsum_t(hh * hh, DH)
    E = _rowsum_t(hh * zh, DH)
    F = _rowsum_t(zh * zh, DH)
    w = dx_ref[...] - (x_ref[...] - zx_ref[...])
    G = _rowsum_t(w * w, 3)
    res = (A - 2.0 * c * B + 2.0 * s * C + c * c * D
           - 2.0 * (c * s) * E + s * s * F + G)
    out_ref[...] = res[None]


def _node_loss(t2, dx_t, dh_t, z_x, z_h, x, h):
    row = lambda i: (i, 0)
    return pl.pallas_call(
        _loss_body,
        grid=(GRID,),
        in_specs=[
            pl.BlockSpec((1, 1, BN), lambda i: (i, 0, 0)),
            pl.BlockSpec((BN, 3), row),
            pl.BlockSpec((BN, DH), row),
            pl.BlockSpec((BN, 3), row),
            pl.BlockSpec((BN, DH), row),
            pl.BlockSpec((BN, 3), row),
            pl.BlockSpec((BN, DH), row),
        ],
        out_specs=pl.BlockSpec((1, 1, BN), lambda i: (i, 0, 0)),
        out_shape=jax.ShapeDtypeStruct((GRID, 1, BN), jnp.float32),
    )(t2, dx_t, dh_t, z_x, z_h, x, h)


def _seg_body(loss_hbm, idx_hbm, out_hbm,
              vals_v, idx_v, ones_v, zeros_v, s_v, c_v, r_v,
              sums_sh, cnts_sh, sem):
    tid = lax.axis_index("s")
    base = tid * PER_TILE

    pltpu.sync_copy(loss_hbm.at[pl.ds(base, PER_TILE)], vals_v)
    pltpu.sync_copy(idx_hbm.at[tid], idx_v)

    for i in range(CH // 16):
        ones_v[pl.ds(16 * i, 16)] = jnp.full((16,), 1.0, jnp.float32)
    for i in range(SLICE // 16):
        zeros_v[pl.ds(16 * i, 16)] = jnp.zeros((16,), jnp.float32)

    pltpu.sync_copy(zeros_v, sums_sh.at[pl.ds(tid * SLICE, SLICE)])
    pltpu.sync_copy(zeros_v, cnts_sh.at[pl.ds(tid * SLICE, SLICE)])
    plsc.subcore_barrier()

    G = 7

    def group(g, carry):
        def fire(j, c2):
            jj = g * G + j
            pltpu.async_copy(vals_v.at[pl.ds(jj * CH, CH)],
                             sums_sh.at[idx_v.at[jj]], sem, add=True)
            pltpu.async_copy(ones_v, cnts_sh.at[idx_v.at[jj]], sem, add=True)
            return c2

        def drain(j, c2):
            pltpu.make_async_copy(ones_v, sums_sh.at[idx_v.at[0]], sem).wait()
            pltpu.make_async_copy(ones_v, cnts_sh.at[idx_v.at[0]], sem).wait()
            return c2

        lax.fori_loop(0, G, fire, 0)
        lax.fori_loop(0, G, drain, 0)
        return carry

    lax.fori_loop(0, NCH // G, group, 0)
    plsc.subcore_barrier()

    pltpu.sync_copy(sums_sh.at[pl.ds(tid * SLICE, SLICE)], s_v)
    pltpu.sync_copy(cnts_sh.at[pl.ds(tid * SLICE, SLICE)], c_v)
    for i in range(SLICE // 16):
        sl = pl.ds(16 * i, 16)
        r_v[sl] = s_v[sl] / jnp.maximum(c_v[sl], 1.0)

    @pl.when(tid < NT - 1)
    def _():
        pltpu.sync_copy(r_v, out_hbm.at[pl.ds(tid * SLICE, SLICE)])

    @pl.when(tid == NT - 1)
    def _():
        rem = NSEG - (NT - 1) * SLICE
        pltpu.sync_copy(r_v.at[pl.ds(0, rem)],
                        out_hbm.at[pl.ds((NT - 1) * SLICE, rem)])


def _seg_call(loss, idx):
    return pl.kernel(
        _seg_body,
        out_type=jax.ShapeDtypeStruct((NSEG,), jnp.float32),
        mesh=plsc.VectorSubcoreMesh(core_axis_name="c", subcore_axis_name="s",
                                    num_cores=1, num_subcores=NT),
        scratch_types=[
        pltpu.VMEM((PER_TILE,), jnp.float32),
        pltpu.VMEM((NCH, CH), jnp.int32),
        pltpu.VMEM((CH,), jnp.float32),
        pltpu.VMEM((SLICE,), jnp.float32),
        pltpu.VMEM((SLICE,), jnp.float32),
        pltpu.VMEM((SLICE,), jnp.float32),
        pltpu.VMEM((SLICE,), jnp.float32),
        pltpu.VMEM_SHARED((NBIN,), jnp.float32),
        pltpu.VMEM_SHARED((NBIN,), jnp.float32),
        pltpu.SemaphoreType.DMA,
        ],
    )(loss, idx)


def kernel(t, dx_t, dh_t, z_x, z_h, x, h, segment_ids):
    t2 = jnp.pad(t.reshape(N), (0, NPAD - N)).reshape(GRID, 1, BN)
    loss = _node_loss(t2, dx_t, dh_t, z_x, z_h, x, h).reshape(NPAD)
    seg = segment_ids.astype(jnp.int32)
    pad = jnp.full((NPAD - N,), NSEG, jnp.int32)
    idx = jnp.concatenate([seg, pad]).reshape(NT, NCH, CH)
    return _seg_call(loss, idx)


# speedup vs baseline: 1.5374x; 1.0298x over previous
import functools
import math

import jax
import jax.numpy as jnp
from jax import lax
from jax.experimental import pallas as pl
from jax.experimental.pallas import tpu as pltpu
from jax.experimental.pallas import tpu_sc as plsc

N = 100000
DH = 128
NSEG = 1000

BN = 7168
GRID = 14
NPAD = BN * GRID

NT = 16
PER_TILE = NPAD // NT
CH = 128
NCH = PER_TILE // CH
NBIN = 1024
SLICE = NBIN // NT

_HALF_PI = 0.5 * math.pi

_COS_C = (0.9999999672669706, -0.49999926887023616, 0.04166409103912293,
          -0.0013857419130200166, 2.3237578000323555e-05)
_SIN_C = (0.9999999827780375, -0.1666665151722809, 0.008332963963586328,
          -0.00019804751716372138, 2.5981027733101847e-06)


def _poly4(u, k):
    acc = jnp.float32(k[4])
    for i in (3, 2, 1, 0):
        acc = acc * u + jnp.float32(k[i])
    return acc


def _rowsum_t(prod, k):
    ones = jnp.ones((k, 1), jnp.float32)
    return jax.lax.dot_general(
        ones, prod, (((0,), (1,)), ((), ())),
        preferred_element_type=jnp.float32)


def _loss_body(t_ref, dx_ref, dh_ref, zx_ref, zh_ref, x_ref, h_ref, out_ref):
    a = _HALF_PI * t_ref[0]
    u = a * a
    c = _HALF_PI * _poly4(u, _COS_C)
    s = (_HALF_PI * a) * _poly4(u, _SIN_C)

    dh = dh_ref[...]
    hh = h_ref[...]
    zh = zh_ref[...]
    A = _rowsum_t(dh * dh, DH)
    B = _rowsum_t(dh * hh, DH)
    C = _rowsum_t(dh * zh, DH)
    D = _rowsum_t(hh * hh, DH)
    E = _rowsum_t(hh * zh, DH)
    F = _rowsum_t(zh * zh, DH)
    w = dx_ref[...] - (x_ref[...] - zx_ref[...])
    G = _rowsum_t(w * w, 3)
    res = (A - 2.0 * c * B + 2.0 * s * C + c * c * D
           - 2.0 * (c * s) * E + s * s * F + G)
    out_ref[...] = res[None]


def _node_loss(t2, dx_t, dh_t, z_x, z_h, x, h):
    row = lambda i: (i, 0)
    return pl.pallas_call(
        _loss_body,
        grid=(GRID,),
        in_specs=[
            pl.BlockSpec((1, 1, BN), lambda i: (i, 0, 0)),
            pl.BlockSpec((BN, 3), row),
            pl.BlockSpec((BN, DH), row),
            pl.BlockSpec((BN, 3), row),
            pl.BlockSpec((BN, DH), row),
            pl.BlockSpec((BN, 3), row),
            pl.BlockSpec((BN, DH), row),
        ],
        out_specs=pl.BlockSpec((1, 1, BN), lambda i: (i, 0, 0)),
        out_shape=jax.ShapeDtypeStruct((GRID, 1, BN), jnp.float32),
    )(t2, dx_t, dh_t, z_x, z_h, x, h)


def _seg_body(loss_hbm, idx_hbm, out_hbm,
              vals_v, idx_v, ones_v, zeros_v, s_v, c_v, r_v,
              sums_sh, cnts_sh, sem):
    tid = lax.axis_index("s")
    base = tid * PER_TILE

    pltpu.sync_copy(loss_hbm.at[pl.ds(base, PER_TILE)], vals_v)
    pltpu.sync_copy(idx_hbm.at[tid], idx_v)

    for i in range(CH // 16):
        ones_v[pl.ds(16 * i, 16)] = jnp.full((16,), 1.0, jnp.float32)
    for i in range(SLICE // 16):
        zeros_v[pl.ds(16 * i, 16)] = jnp.zeros((16,), jnp.float32)

    pltpu.sync_copy(zeros_v, sums_sh.at[pl.ds(tid * SLICE, SLICE)])
    pltpu.sync_copy(zeros_v, cnts_sh.at[pl.ds(tid * SLICE, SLICE)])
    plsc.subcore_barrier()

    G = 7

    def group(g, carry):
        def fire(j, c2):
            jj = g * G + j
            pltpu.async_copy(vals_v.at[pl.ds(jj * CH, CH)],
                             sums_sh.at[idx_v.at[jj]], sem, add=True)
            pltpu.async_copy(ones_v, cnts_sh.at[idx_v.at[jj]], sem, add=True)
            return c2

        def drain(j, c2):
            pltpu.make_async_copy(ones_v, sums_sh.at[idx_v.at[0]], sem).wait()
            pltpu.make_async_copy(ones_v, cnts_sh.at[idx_v.at[0]], sem).wait()
            return c2

        lax.fori_loop(0, G, fire, 0)
        lax.fori_loop(0, G, drain, 0)
        return carry

    lax.fori_loop(0, NCH // G, group, 0)
    plsc.subcore_barrier()

    pltpu.sync_copy(sums_sh.at[pl.ds(tid * SLICE, SLICE)], s_v)
    pltpu.sync_copy(cnts_sh.at[pl.ds(tid * SLICE, SLICE)], c_v)
    for i in range(SLICE // 16):
        sl = pl.ds(16 * i, 16)
        r_v[sl] = s_v[sl] / jnp.maximum(c_v[sl], 1.0)

    @pl.when(tid < NT - 1)
    def _():
        pltpu.sync_copy(r_v, out_hbm.at[pl.ds(tid * SLICE, SLICE)])

    @pl.when(tid == NT - 1)
    def _():
        rem = NSEG - (NT - 1) * SLICE
        pltpu.sync_copy(r_v.at[pl.ds(0, rem)],
                        out_hbm.at[pl.ds((NT - 1) * SLICE, rem)])


def _seg_call(loss, idx):
    return pl.kernel(
        _seg_body,
        out_type=jax.ShapeDtypeStruct((NSEG,), jnp.float32),
        mesh=plsc.VectorSubcoreMesh(core_axis_name="c", subcore_axis_name="s",
                                    num_cores=1, num_subcores=NT),
        scratch_types=[
        pltpu.VMEM((PER_TILE,), jnp.float32),
        pltpu.VMEM((NCH, CH), jnp.int32),
        pltpu.VMEM((CH,), jnp.float32),
        pltpu.VMEM((SLICE,), jnp.float32),
        pltpu.VMEM((SLICE,), jnp.float32),
        pltpu.VMEM((SLICE,), jnp.float32),
        pltpu.VMEM((SLICE,), jnp.float32),
        pltpu.VMEM_SHARED((NBIN,), jnp.float32),
        pltpu.VMEM_SHARED((NBIN,), jnp.float32),
        pltpu.SemaphoreType.DMA,
        ],
    )(loss, idx)


def kernel(t, dx_t, dh_t, z_x, z_h, x, h, segment_ids):
    t2 = jnp.pad(t.reshape(N), (0, NPAD - N)).reshape(GRID, 1, BN)
    loss = _node_loss(t2, dx_t, dh_t, z_x, z_h, x, h).reshape(NPAD)
    seg = segment_ids.astype(jnp.int32)
    pad = jnp.full((NPAD - N,), NSEG, jnp.int32)
    idx = jnp.concatenate([seg, pad]).reshape(NT, NCH, CH)
    return _seg_call(loss, idx)
